# baseline (speedup 1.0000x reference)
import jax
import jax.numpy as jnp
from jax import lax
from jax.experimental import pallas as pl
from jax.experimental.pallas import tpu as pltpu

_BATCH_TILE = 4096
_SPLIT = 2


def _linear_kernel(x_ref, w_ref, b_ref, o_ref):
    j = pl.program_id(1)
    t = o_ref.shape[0]
    acc = lax.dot_general(
        x_ref[pl.ds(j * t, t), :], w_ref[...],
        dimension_numbers=(((1,), (1,)), ((), ())),
        preferred_element_type=jnp.float32,
    )
    o_ref[...] = (acc + b_ref[...]).astype(o_ref.dtype)


def kernel(x, w, b):
    batch, input_dim = x.shape
    output_dim = w.shape[0]
    b2 = b.reshape(1, output_dim).astype(jnp.float32)

    tile = min(_BATCH_TILE, batch)
    sub = tile // _SPLIT

    cost = pl.CostEstimate(
        flops=2 * batch * input_dim * output_dim,
        transcendentals=0,
        bytes_accessed=(x.size * 4 + w.size * 4 + b2.size * 4
                       + batch * output_dim * 4),
    )
    return pl.pallas_call(
        _linear_kernel,
        out_shape=jax.ShapeDtypeStruct((batch, output_dim), jnp.float32),
        grid=(pl.cdiv(batch, tile), _SPLIT),
        in_specs=[
            pl.BlockSpec((tile, input_dim), lambda i, j: (i, 0)),
            pl.BlockSpec((output_dim, input_dim), lambda i, j: (0, 0)),
            pl.BlockSpec((1, output_dim), lambda i, j: (0, 0)),
        ],
        out_specs=pl.BlockSpec((sub, output_dim),
                               lambda i, j: (i * _SPLIT + j, 0)),
        compiler_params=pltpu.CompilerParams(
            dimension_semantics=("parallel", "arbitrary"),
            vmem_limit_bytes=64 << 20,
        ),
        cost_estimate=cost,
    )(x, w, b2)


# speedup vs baseline: 1.3174x; 1.3174x over previous
import jax
import jax.numpy as jnp
from jax import lax
from jax.experimental import pallas as pl
from jax.experimental.pallas import tpu as pltpu

_BATCH_TILE = 4096


def _linear_kernel(x_ref, w_ref, b_ref, o_ref):
    acc = lax.dot_general(
        x_ref[...], w_ref[...],
        dimension_numbers=(((1,), (1,)), ((), ())),
        preferred_element_type=jnp.float32,
    )
    o_ref[...] = (acc + b_ref[...]).astype(o_ref.dtype)


def kernel(x, w, b):
    batch, input_dim = x.shape
    output_dim = w.shape[0]
    b2 = b.reshape(1, output_dim).astype(jnp.float32)

    tile = min(_BATCH_TILE, batch)

    cost = pl.CostEstimate(
        flops=2 * batch * input_dim * output_dim,
        transcendentals=0,
        bytes_accessed=(x.size * 4 + w.size * 4 + b2.size * 4
                       + batch * output_dim * 4),
    )
    return pl.pallas_call(
        _linear_kernel,
        out_shape=jax.ShapeDtypeStruct((batch, output_dim), jnp.float32),
        grid=(pl.cdiv(batch, tile),),
        in_specs=[
            pl.BlockSpec((tile, input_dim), lambda i: (i, 0)),
            pl.BlockSpec((output_dim, input_dim), lambda i: (0, 0)),
            pl.BlockSpec((1, output_dim), lambda i: (0, 0)),
        ],
        out_specs=pl.BlockSpec((tile, output_dim), lambda i: (i, 0)),
        compiler_params=pltpu.CompilerParams(
            dimension_semantics=("parallel",),
            vmem_limit_bytes=64 << 20,
        ),
        cost_estimate=cost,
    )(x, w, b2)
